# baseline (speedup 1.0000x reference)
import jax
import jax.numpy as jnp
from jax.experimental import pallas as pl
from jax.experimental.pallas import tpu as pltpu

_D = 4096
_E = 64
_K = 8
_TB = 1024
_NC = 2


def _route_chunk(probs):
    tc = probs.shape[0]
    lane = jax.lax.broadcasted_iota(jnp.int32, (tc, _E), 1)
    lane_k = jax.lax.broadcasted_iota(jnp.int32, (tc, _K), 1)
    work = probs
    gw = jnp.zeros((tc, _E), jnp.float32)
    idx_out = jnp.zeros((tc, _K), jnp.int32)
    for k in range(_K):
        idx = jnp.argmax(work, axis=-1, keepdims=True)
        onehot = lane == idx
        gw = jnp.where(onehot, work, gw)
        idx_out = jnp.where(lane_k == k, idx, idx_out)
        work = jnp.where(onehot, -1.0, work)
    ssum = jnp.sum(gw, axis=-1, keepdims=True)
    return gw / ssum, idx_out


def _router_kernel(x_ref, w_ref, gw_ref, idx_ref, fpart_ref, ppart_ref):
    w = w_ref[...]
    tc = _TB // _NC
    facc = jnp.zeros((1, _E), jnp.float32)
    pacc = jnp.zeros((1, _E), jnp.float32)
    for c in range(_NC):
        sl = pl.ds(c * tc, tc)
        logits = jnp.dot(x_ref[sl, :], w, preferred_element_type=jnp.float32)
        m = jnp.max(logits, axis=-1, keepdims=True)
        e = jnp.exp(logits - m)
        probs = e / jnp.sum(e, axis=-1, keepdims=True)
        gw, idx_out = _route_chunk(probs)
        gw_ref[sl, :] = gw
        idx_ref[sl, :] = idx_out
        pacc += jnp.sum(probs, axis=0, keepdims=True)
        facc += jnp.sum((gw > 0.0).astype(jnp.float32), axis=0, keepdims=True)
    fpart_ref[...] = facc.reshape(1, 1, _E)
    ppart_ref[...] = pacc.reshape(1, 1, _E)


def _aux_kernel(fpart_ref, ppart_ref, aux_ref, *, t_total):
    f = jnp.sum(fpart_ref[...], axis=0) / (t_total * _K)
    p = jnp.sum(ppart_ref[...], axis=0) / t_total
    aux_ref[...] = (_E * jnp.sum(f * p)).reshape(1, 1)


import functools


def kernel(x, W_gate):
    t = x.shape[0]
    grid = t // _TB
    gw, idx, fpart, ppart = pl.pallas_call(
        _router_kernel,
        grid=(grid,),
        in_specs=[
            pl.BlockSpec((_TB, _D), lambda i: (i, 0)),
            pl.BlockSpec((_D, _E), lambda i: (0, 0)),
        ],
        out_specs=[
            pl.BlockSpec((_TB, _E), lambda i: (i, 0)),
            pl.BlockSpec((_TB, _K), lambda i: (i, 0)),
            pl.BlockSpec((1, 1, _E), lambda i: (i, 0, 0)),
            pl.BlockSpec((1, 1, _E), lambda i: (i, 0, 0)),
        ],
        out_shape=[
            jax.ShapeDtypeStruct((t, _E), jnp.float32),
            jax.ShapeDtypeStruct((t, _K), jnp.int32),
            jax.ShapeDtypeStruct((grid, 1, _E), jnp.float32),
            jax.ShapeDtypeStruct((grid, 1, _E), jnp.float32),
        ],
        compiler_params=pltpu.CompilerParams(
            dimension_semantics=("parallel",),
        ),
    )(x, W_gate)
    aux = pl.pallas_call(
        functools.partial(_aux_kernel, t_total=float(t)),
        out_shape=jax.ShapeDtypeStruct((1, 1), jnp.float32),
    )(fpart.reshape(grid, _E), ppart.reshape(grid, _E))
    return gw, idx, aux[0, 0]


# speedup vs baseline: 1.0072x; 1.0072x over previous
import jax
import jax.numpy as jnp
from jax.experimental import pallas as pl
from jax.experimental.pallas import tpu as pltpu

_D = 4096
_E = 64
_K = 8
_TBH = 512
_TB = 2 * _TBH


def _route_chunk(probs):
    tc = probs.shape[0]
    lane = jax.lax.broadcasted_iota(jnp.int32, (tc, _E), 1)
    lane_k = jax.lax.broadcasted_iota(jnp.int32, (tc, _K), 1)
    work = probs
    gw = jnp.zeros((tc, _E), jnp.float32)
    idx_out = jnp.zeros((tc, _K), jnp.int32)
    for k in range(_K):
        idx = jnp.argmax(work, axis=-1, keepdims=True)
        onehot = lane == idx
        gw = jnp.where(onehot, work, gw)
        idx_out = jnp.where(lane_k == k, idx, idx_out)
        work = jnp.where(onehot, -1.0, work)
    ssum = jnp.sum(gw, axis=-1, keepdims=True)
    return gw / ssum, idx_out


def _router_kernel(xa_ref, xb_ref, w_ref, gw_ref, idx_ref, aux_ref,
                   fsum_ref, psum_ref):
    i = pl.program_id(0)
    n = pl.num_programs(0)

    @pl.when(i == 0)
    def _init():
        fsum_ref[...] = jnp.zeros_like(fsum_ref)
        psum_ref[...] = jnp.zeros_like(psum_ref)

    w = w_ref[...]
    facc = jnp.zeros((1, _E), jnp.float32)
    pacc = jnp.zeros((1, _E), jnp.float32)
    for c, x_ref in enumerate((xa_ref, xb_ref)):
        sl = pl.ds(c * _TBH, _TBH)
        logits = jnp.dot(x_ref[...], w, preferred_element_type=jnp.float32)
        m = jnp.max(logits, axis=-1, keepdims=True)
        e = jnp.exp(logits - m)
        probs = e / jnp.sum(e, axis=-1, keepdims=True)
        gw, idx_out = _route_chunk(probs)
        gw_ref[sl, :] = gw
        idx_ref[sl, :] = idx_out
        pacc += jnp.sum(probs, axis=0, keepdims=True)
        facc += jnp.sum((gw > 0.0).astype(jnp.float32), axis=0, keepdims=True)
    fsum_ref[...] += facc
    psum_ref[...] += pacc

    @pl.when(i == n - 1)
    def _final():
        t_total = jnp.float32(n * _TB)
        f = fsum_ref[...] / (t_total * _K)
        p = psum_ref[...] / t_total
        aux_ref[...] = (_E * jnp.sum(f * p)).reshape(1, 1)


def kernel(x, W_gate):
    t = x.shape[0]
    grid = t // _TB
    gw, idx, aux = pl.pallas_call(
        _router_kernel,
        grid=(grid,),
        in_specs=[
            pl.BlockSpec((_TBH, _D), lambda i: (2 * i, 0)),
            pl.BlockSpec((_TBH, _D), lambda i: (2 * i + 1, 0)),
            pl.BlockSpec((_D, _E), lambda i: (0, 0)),
        ],
        out_specs=[
            pl.BlockSpec((_TB, _E), lambda i: (i, 0)),
            pl.BlockSpec((_TB, _K), lambda i: (i, 0)),
            pl.BlockSpec((1, 1), lambda i: (0, 0)),
        ],
        out_shape=[
            jax.ShapeDtypeStruct((t, _E), jnp.float32),
            jax.ShapeDtypeStruct((t, _K), jnp.int32),
            jax.ShapeDtypeStruct((1, 1), jnp.float32),
        ],
        scratch_shapes=[
            pltpu.VMEM((1, _E), jnp.float32),
            pltpu.VMEM((1, _E), jnp.float32),
        ],
    )(x, x, W_gate)
    return gw, idx, aux[0, 0]


# speedup vs baseline: 1.0183x; 1.0110x over previous
import jax
import jax.numpy as jnp
from jax.experimental import pallas as pl
from jax.experimental.pallas import tpu as pltpu

_D = 4096
_E = 64
_K = 8
_TB = 1024
_NC = 2


def _route_chunk(probs):
    tc = probs.shape[0]
    lane = jax.lax.broadcasted_iota(jnp.int32, (tc, _E), 1)
    lane_k = jax.lax.broadcasted_iota(jnp.int32, (tc, _K), 1)
    work = probs
    gw = jnp.zeros((tc, _E), jnp.float32)
    idx_out = jnp.zeros((tc, _K), jnp.int32)
    for k in range(_K):
        idx = jnp.argmax(work, axis=-1, keepdims=True)
        onehot = lane == idx
        gw = jnp.where(onehot, work, gw)
        idx_out = jnp.where(lane_k == k, idx, idx_out)
        work = jnp.where(onehot, -1.0, work)
    ssum = jnp.sum(gw, axis=-1, keepdims=True)
    return gw / ssum, idx_out


def _router_kernel(x_ref, w_ref, gw_ref, idx_ref, aux_ref, fsum_ref, psum_ref):
    i = pl.program_id(0)
    n = pl.num_programs(0)

    @pl.when(i == 0)
    def _init():
        fsum_ref[...] = jnp.zeros_like(fsum_ref)
        psum_ref[...] = jnp.zeros_like(psum_ref)

    w = w_ref[...]
    tc = _TB // _NC
    facc = jnp.zeros((1, _E), jnp.float32)
    pacc = jnp.zeros((1, _E), jnp.float32)
    for c in range(_NC):
        sl = pl.ds(c * tc, tc)
        logits = jnp.dot(x_ref[sl, :], w, preferred_element_type=jnp.float32)
        m = jnp.max(logits, axis=-1, keepdims=True)
        e = jnp.exp(logits - m)
        probs = e / jnp.sum(e, axis=-1, keepdims=True)
        gw, idx_out = _route_chunk(probs)
        gw_ref[sl, :] = gw
        idx_ref[sl, :] = idx_out
        pacc += jnp.sum(probs, axis=0, keepdims=True)
        facc += jnp.sum((gw > 0.0).astype(jnp.float32), axis=0, keepdims=True)
    fsum_ref[...] += facc
    psum_ref[...] += pacc

    @pl.when(i == n - 1)
    def _final():
        t_total = jnp.float32(n * _TB)
        f = fsum_ref[...] / (t_total * _K)
        p = psum_ref[...] / t_total
        aux_ref[...] = (_E * jnp.sum(f * p)).reshape(1, 1)


def kernel(x, W_gate):
    t = x.shape[0]
    grid = t // _TB
    gw, idx, aux = pl.pallas_call(
        _router_kernel,
        grid=(grid,),
        in_specs=[
            pl.BlockSpec((_TB, _D), lambda i: (i, 0)),
            pl.BlockSpec((_D, _E), lambda i: (0, 0)),
        ],
        out_specs=[
            pl.BlockSpec((_TB, _E), lambda i: (i, 0)),
            pl.BlockSpec((_TB, _K), lambda i: (i, 0)),
            pl.BlockSpec((1, 1), lambda i: (0, 0)),
        ],
        out_shape=[
            jax.ShapeDtypeStruct((t, _E), jnp.float32),
            jax.ShapeDtypeStruct((t, _K), jnp.int32),
            jax.ShapeDtypeStruct((1, 1), jnp.float32),
        ],
        scratch_shapes=[
            pltpu.VMEM((1, _E), jnp.float32),
            pltpu.VMEM((1, _E), jnp.float32),
        ],
    )(x, W_gate)
    return gw, idx, aux[0, 0]


# speedup vs baseline: 1.0264x; 1.0079x over previous
import jax
import jax.numpy as jnp
from jax.experimental import pallas as pl
from jax.experimental.pallas import tpu as pltpu

_D = 4096
_E = 64
_K = 8
_TB = 1024
_NC = 4


def _route_chunk(probs):
    tc = probs.shape[0]
    lane = jax.lax.broadcasted_iota(jnp.int32, (tc, _E), 1)
    lane_k = jax.lax.broadcasted_iota(jnp.int32, (tc, _K), 1)
    work = probs
    gw = jnp.zeros((tc, _E), jnp.float32)
    idx_out = jnp.zeros((tc, _K), jnp.int32)
    for k in range(_K):
        idx = jnp.argmax(work, axis=-1, keepdims=True)
        onehot = lane == idx
        gw = jnp.where(onehot, work, gw)
        idx_out = jnp.where(lane_k == k, idx, idx_out)
        work = jnp.where(onehot, -1.0, work)
    ssum = jnp.sum(gw, axis=-1, keepdims=True)
    return gw / ssum, idx_out


def _router_kernel(x_ref, w_ref, gw_ref, idx_ref, aux_ref, fsum_ref, psum_ref):
    i = pl.program_id(0)
    n = pl.num_programs(0)

    @pl.when(i == 0)
    def _init():
        fsum_ref[...] = jnp.zeros_like(fsum_ref)
        psum_ref[...] = jnp.zeros_like(psum_ref)

    w = w_ref[...]
    tc = _TB // _NC
    facc = jnp.zeros((1, _E), jnp.float32)
    pacc = jnp.zeros((1, _E), jnp.float32)
    for c in range(_NC):
        sl = pl.ds(c * tc, tc)
        logits = jnp.dot(x_ref[sl, :], w, preferred_element_type=jnp.float32)
        m = jnp.max(logits, axis=-1, keepdims=True)
        e = jnp.exp(logits - m)
        probs = e / jnp.sum(e, axis=-1, keepdims=True)
        gw, idx_out = _route_chunk(probs)
        gw_ref[sl, :] = gw
        idx_ref[sl, :] = idx_out
        pacc += jnp.sum(probs, axis=0, keepdims=True)
        facc += jnp.sum((gw > 0.0).astype(jnp.float32), axis=0, keepdims=True)
    fsum_ref[...] += facc
    psum_ref[...] += pacc

    @pl.when(i == n - 1)
    def _final():
        t_total = jnp.float32(n * _TB)
        f = fsum_ref[...] / (t_total * _K)
        p = psum_ref[...] / t_total
        aux_ref[...] = (_E * jnp.sum(f * p)).reshape(1, 1)


def kernel(x, W_gate):
    t = x.shape[0]
    grid = t // _TB
    gw, idx, aux = pl.pallas_call(
        _router_kernel,
        grid=(grid,),
        in_specs=[
            pl.BlockSpec((_TB, _D), lambda i: (i, 0)),
            pl.BlockSpec((_D, _E), lambda i: (0, 0)),
        ],
        out_specs=[
            pl.BlockSpec((_TB, _E), lambda i: (i, 0)),
            pl.BlockSpec((_TB, _K), lambda i: (i, 0)),
            pl.BlockSpec((1, 1), lambda i: (0, 0)),
        ],
        out_shape=[
            jax.ShapeDtypeStruct((t, _E), jnp.float32),
            jax.ShapeDtypeStruct((t, _K), jnp.int32),
            jax.ShapeDtypeStruct((1, 1), jnp.float32),
        ],
        scratch_shapes=[
            pltpu.VMEM((1, _E), jnp.float32),
            pltpu.VMEM((1, _E), jnp.float32),
        ],
    )(x, W_gate)
    return gw, idx, aux[0, 0]
